# Initial kernel scaffold; baseline (speedup 1.0000x reference)
import jax
import jax.numpy as jnp
from jax.experimental import pallas as pl


def kernel(x, neighborhood):
    raise NotImplementedError("write your pallas kernel here")


# speedup vs baseline: 2.4202x; 2.4202x over previous
import functools

import jax
import jax.numpy as jnp
import numpy as np
from jax import lax
from jax.experimental import pallas as pl
from jax.experimental.pallas import tpu as pltpu
from jax.experimental.pallas import tpu_sc as plsc

LANES = 16
NUM_CORES = 2
NUM_SUBCORES = 16
NUM_WORKERS = NUM_CORES * NUM_SUBCORES
TILE_R = 64


def _kept_node_indices(n_in: int, n_feat: int) -> np.ndarray:
    t_in, t_out = 64, 32
    step = t_in // t_out
    chunk = n_in // t_in
    keep_per_t = 128 // t_out
    time_indices = range(0, t_in, step)
    return np.array(
        [s * chunk + i for s in time_indices for i in range(keep_per_t)],
        dtype=np.int32,
    )


@functools.partial(jax.jit, static_argnames=("rows", "n_in", "n_out", "n_nbr"))
def _sc_pool(x2, nbr_t, *, rows, n_in, n_out, n_nbr):
    rows_per_w = rows // NUM_WORKERS
    n_tiles = rows_per_w // TILE_R
    n_chunks = n_out // LANES

    mesh = plsc.VectorSubcoreMesh(core_axis_name="c", subcore_axis_name="s")

    @functools.partial(
        pl.kernel,
        mesh=mesh,
        compiler_params=pltpu.CompilerParams(
            use_tc_tiling_on_sc=False, needs_layout_passes=False
        ),
        out_type=jax.ShapeDtypeStruct((rows, n_out), jnp.float32),
        scratch_types=[
            pltpu.VMEM((TILE_R, n_in), jnp.float32),
            pltpu.VMEM((n_nbr, n_out), jnp.int32),
            pltpu.VMEM((TILE_R, n_out), jnp.float32),
        ],
    )
    def k(x_hbm, nbr_hbm, out_hbm, x_v, nbr_v, out_v):
        wid = lax.axis_index("s") * NUM_CORES + lax.axis_index("c")
        pltpu.sync_copy(nbr_hbm, nbr_v)
        for t in range(n_tiles):
            row0 = wid * rows_per_w + t * TILE_R
            pltpu.sync_copy(x_hbm.at[pl.ds(row0, TILE_R), :], x_v)

            def body(r, carry):
                rsplat = jnp.full((LANES,), r, dtype=jnp.int32)
                for c in range(n_chunks):
                    acc = plsc.load_gather(
                        x_v, [rsplat, nbr_v[0, pl.ds(c * LANES, LANES)]]
                    )
                    for j in range(1, n_nbr):
                        acc = jnp.maximum(
                            acc,
                            plsc.load_gather(
                                x_v, [rsplat, nbr_v[j, pl.ds(c * LANES, LANES)]]
                            ),
                        )
                    out_v[r, pl.ds(c * LANES, LANES)] = acc
                return carry

            lax.fori_loop(0, TILE_R, body, 0)
            pltpu.sync_copy(out_v, out_hbm.at[pl.ds(row0, TILE_R), :])

    return k(x2, nbr_t)


def kernel(x, neighborhood):
    b, f, n_in = x.shape
    n_nbr = neighborhood.shape[1]
    keep = _kept_node_indices(n_in, f)
    n_out = keep.shape[0]
    nbr_t = neighborhood[keep, :].T.astype(jnp.int32)
    x2 = x.reshape(b * f, n_in)
    out = _sc_pool(
        x2, nbr_t, rows=b * f, n_in=n_in, n_out=n_out, n_nbr=n_nbr
    )
    return out.reshape(b, f, n_out)


# speedup vs baseline: 3.0352x; 1.2541x over previous
import functools

import jax
import jax.numpy as jnp
import numpy as np
from jax import lax
from jax.experimental import pallas as pl
from jax.experimental.pallas import tpu as pltpu
from jax.experimental.pallas import tpu_sc as plsc

LANES = 16
NUM_CORES = 2
NUM_SUBCORES = 16
NUM_WORKERS = NUM_CORES * NUM_SUBCORES
TILE_R = 64


def _kept_node_indices(n_in: int) -> np.ndarray:
    t_in, t_out = 64, 32
    step = t_in // t_out
    chunk = n_in // t_in
    keep_per_t = 128 // t_out
    time_indices = range(0, t_in, step)
    return np.array(
        [s * chunk + i for s in time_indices for i in range(keep_per_t)],
        dtype=np.int32,
    )


@functools.partial(jax.jit, static_argnames=("rows", "n_in", "n_out", "n_nbr"))
def _sc_pool(x2, nbr_t, *, rows, n_in, n_out, n_nbr):
    rows_per_w = rows // NUM_WORKERS
    n_tiles = rows_per_w // TILE_R
    n_chunks = n_out // LANES

    mesh = plsc.VectorSubcoreMesh(core_axis_name="c", subcore_axis_name="s")

    @functools.partial(
        pl.kernel,
        mesh=mesh,
        compiler_params=pltpu.CompilerParams(
            use_tc_tiling_on_sc=False, needs_layout_passes=False
        ),
        out_type=jax.ShapeDtypeStruct((rows, n_out), jnp.float32),
        scratch_types=[
            pltpu.VMEM((TILE_R, n_in), jnp.float32),
            pltpu.VMEM((TILE_R, n_in), jnp.float32),
            pltpu.VMEM((TILE_R, n_out), jnp.float32),
            pltpu.VMEM((TILE_R, n_out), jnp.float32),
            pltpu.VMEM((n_nbr, n_out), jnp.int32),
            pltpu.SemaphoreType.DMA,
            pltpu.SemaphoreType.DMA,
            pltpu.SemaphoreType.DMA,
            pltpu.SemaphoreType.DMA,
        ],
    )
    def k(x_hbm, nbr_hbm, out_hbm, xv0, xv1, ov0, ov1, nbr_v, si0, si1, so0, so1):
        wid = lax.axis_index("s") * NUM_CORES + lax.axis_index("c")
        base = wid * rows_per_w
        xv = (xv0, xv1)
        ov = (ov0, ov1)
        sin = (si0, si1)
        sout = (so0, so1)
        pltpu.sync_copy(nbr_hbm, nbr_v)

        def start_in(t):
            buf = t % 2
            return pltpu.async_copy(
                x_hbm.at[pl.ds(base + t * TILE_R, TILE_R), :], xv[buf], sin[buf]
            )

        in_copies = [None] * n_tiles
        out_copies = [None] * n_tiles
        in_copies[0] = start_in(0)
        for t in range(n_tiles):
            buf = t % 2
            if t + 1 < n_tiles:
                in_copies[t + 1] = start_in(t + 1)
            in_copies[t].wait()
            if t >= 2:
                out_copies[t - 2].wait()
            xb, ob = xv[buf], ov[buf]
            for c in range(n_chunks):
                idxs = [nbr_v[j, pl.ds(c * LANES, LANES)] for j in range(n_nbr)]

                def body(r, carry, idxs=idxs, xb=xb, ob=ob, c=c):
                    rsplat = jnp.full((LANES,), r, dtype=jnp.int32)
                    acc = plsc.load_gather(xb, [rsplat, idxs[0]])
                    for j in range(1, n_nbr):
                        acc = jnp.maximum(
                            acc, plsc.load_gather(xb, [rsplat, idxs[j]])
                        )
                    ob[r, pl.ds(c * LANES, LANES)] = acc
                    return carry

                lax.fori_loop(0, TILE_R, body, 0)
            out_copies[t] = pltpu.async_copy(
                ob, out_hbm.at[pl.ds(base + t * TILE_R, TILE_R), :], sout[buf]
            )
        for t in range(max(0, n_tiles - 2), n_tiles):
            out_copies[t].wait()

    return k(x2, nbr_t)


def kernel(x, neighborhood):
    b, f, n_in = x.shape
    n_nbr = neighborhood.shape[1]
    keep = _kept_node_indices(n_in)
    n_out = keep.shape[0]
    nbr_t = neighborhood[keep, :].T.astype(jnp.int32)
    x2 = x.reshape(b * f, n_in)
    out = _sc_pool(
        x2, nbr_t, rows=b * f, n_in=n_in, n_out=n_out, n_nbr=n_nbr
    )
    return out.reshape(b, f, n_out)


# speedup vs baseline: 3.8693x; 1.2748x over previous
import functools

import jax
import jax.numpy as jnp
import numpy as np
from jax import lax
from jax.experimental import pallas as pl
from jax.experimental.pallas import tpu as pltpu
from jax.experimental.pallas import tpu_sc as plsc

LANES = 16
NUM_CORES = 2
NUM_SUBCORES = 16
NUM_WORKERS = NUM_CORES * NUM_SUBCORES
TILE_R = 64


def _kept_node_indices(n_in: int) -> np.ndarray:
    t_in, t_out = 64, 32
    step = t_in // t_out
    chunk = n_in // t_in
    keep_per_t = 128 // t_out
    time_indices = range(0, t_in, step)
    return np.array(
        [s * chunk + i for s in time_indices for i in range(keep_per_t)],
        dtype=np.int32,
    )


@functools.partial(jax.jit, static_argnames=("rows", "n_in", "n_out", "n_nbr"))
def _sc_pool(x2, nbr_t, *, rows, n_in, n_out, n_nbr):
    rows_per_w = rows // NUM_WORKERS
    n_tiles = rows_per_w // TILE_R
    n_chunks = n_out // LANES

    mesh = plsc.VectorSubcoreMesh(core_axis_name="c", subcore_axis_name="s")

    @functools.partial(
        pl.kernel,
        mesh=mesh,
        compiler_params=pltpu.CompilerParams(
            use_tc_tiling_on_sc=True, needs_layout_passes=False
        ),
        out_type=jax.ShapeDtypeStruct((rows, n_out), jnp.float32),
        scratch_types=[
            pltpu.VMEM((TILE_R, n_in), jnp.float32),
            pltpu.VMEM((TILE_R, n_in), jnp.float32),
            pltpu.VMEM((TILE_R, n_out), jnp.float32),
            pltpu.VMEM((TILE_R, n_out), jnp.float32),
            pltpu.VMEM((n_nbr, n_out), jnp.int32),
            pltpu.SemaphoreType.DMA,
            pltpu.SemaphoreType.DMA,
            pltpu.SemaphoreType.DMA,
            pltpu.SemaphoreType.DMA,
        ],
    )
    def k(x_hbm, nbr_hbm, out_hbm, xv0, xv1, ov0, ov1, nbr_v, si0, si1, so0, so1):
        wid = lax.axis_index("s") * NUM_CORES + lax.axis_index("c")
        base = wid * rows_per_w
        xv = (xv0, xv1)
        ov = (ov0, ov1)
        sin = (si0, si1)
        sout = (so0, so1)
        pltpu.sync_copy(nbr_hbm, nbr_v)

        def start_in(t):
            buf = t % 2
            return pltpu.async_copy(
                x_hbm.at[pl.ds(base + t * TILE_R, TILE_R), :], xv[buf], sin[buf]
            )

        in_copies = [None] * n_tiles
        out_copies = [None] * n_tiles
        in_copies[0] = start_in(0)
        for t in range(n_tiles):
            buf = t % 2
            if t + 1 < n_tiles:
                in_copies[t + 1] = start_in(t + 1)
            in_copies[t].wait()
            if t >= 2:
                out_copies[t - 2].wait()
            xb, ob = xv[buf], ov[buf]
            for c in range(n_chunks):
                idxs = [nbr_v[j, pl.ds(c * LANES, LANES)] for j in range(n_nbr)]

                def body(r, carry, idxs=idxs, xb=xb, ob=ob, c=c):
                    rsplat = jnp.full((LANES,), r, dtype=jnp.int32)
                    acc = plsc.load_gather(xb, [rsplat, idxs[0]])
                    for j in range(1, n_nbr):
                        acc = jnp.maximum(
                            acc, plsc.load_gather(xb, [rsplat, idxs[j]])
                        )
                    ob[r, pl.ds(c * LANES, LANES)] = acc
                    return carry

                lax.fori_loop(0, TILE_R, body, 0)
            out_copies[t] = pltpu.async_copy(
                ob, out_hbm.at[pl.ds(base + t * TILE_R, TILE_R), :], sout[buf]
            )
        for t in range(max(0, n_tiles - 2), n_tiles):
            out_copies[t].wait()

    return k(x2, nbr_t)


def kernel(x, neighborhood):
    b, f, n_in = x.shape
    n_nbr = neighborhood.shape[1]
    keep = _kept_node_indices(n_in)
    n_out = keep.shape[0]
    nbr_t = neighborhood[keep, :].T.astype(jnp.int32)
    x2 = x.reshape(b * f, n_in)
    out = _sc_pool(
        x2, nbr_t, rows=b * f, n_in=n_in, n_out=n_out, n_nbr=n_nbr
    )
    return out.reshape(b, f, n_out)
